# Initial kernel scaffold; baseline (speedup 1.0000x reference)
import jax
import jax.numpy as jnp
from jax.experimental import pallas as pl


def kernel(stu_id, input_exercise, inut_word, inut_format, inut_section, inut_wordlen, inut_cefr, input_knowledge_point, student_emb, k_difficulty, e_difficulty, k_difficulty_i, e_difficulty_i, a_w1, a_b1, a_w2, a_b2, b_w1, b_b1, b_w2, b_b2, t_w1, t_b1, t_w2, t_b2):
    raise NotImplementedError("write your pallas kernel here")


# speedup vs baseline: 5.4301x; 5.4301x over previous
import functools

import jax
import jax.numpy as jnp
from jax import lax
from jax.experimental import pallas as pl
from jax.experimental.pallas import tpu as pltpu
from jax.experimental.pallas import tpu_sc as plsc


def _gather3(student_emb, k_difficulty, e_difficulty, stu_id, inut_word):
    B = stu_id.shape[0]
    K = student_emb.shape[1]
    info = plsc.get_sparse_core_info()
    nw = info.num_cores * info.num_subcores
    b_per_w = B // nw
    mesh = plsc.VectorSubcoreMesh(core_axis_name="c", subcore_axis_name="s")

    @functools.partial(
        pl.kernel,
        mesh=mesh,
        out_type=(
            jax.ShapeDtypeStruct((B, K), jnp.float32),
            jax.ShapeDtypeStruct((B, K), jnp.float32),
            jax.ShapeDtypeStruct((B, K), jnp.float32),
        ),
        scratch_types=[
            pltpu.VMEM((b_per_w,), jnp.int32),
            pltpu.VMEM((b_per_w, K), jnp.float32),
            pltpu.SemaphoreType.DMA,
        ],
    )
    def gather_kernel(stu_hbm, kd_hbm, ed_hbm, sid_hbm, wid_hbm,
                      out_s, out_k, out_e, idx_v, rows_v, sem):
        w = lax.axis_index("s") * info.num_cores + lax.axis_index("c")
        base = w * b_per_w
        pltpu.sync_copy(sid_hbm.at[pl.ds(base, b_per_w)], idx_v)
        pltpu.async_copy(stu_hbm.at[idx_v], rows_v, sem).wait()
        pltpu.sync_copy(rows_v, out_s.at[pl.ds(base, b_per_w)])
        pltpu.sync_copy(wid_hbm.at[pl.ds(base, b_per_w)], idx_v)
        pltpu.async_copy(kd_hbm.at[idx_v], rows_v, sem).wait()
        pltpu.sync_copy(rows_v, out_k.at[pl.ds(base, b_per_w)])
        pltpu.async_copy(ed_hbm.at[idx_v], rows_v, sem).wait()
        pltpu.sync_copy(rows_v, out_e.at[pl.ds(base, b_per_w)])

    return gather_kernel(student_emb, k_difficulty, e_difficulty,
                         stu_id.astype(jnp.int32), inut_word.astype(jnp.int32))


def _mlp_body(stu_ref, e_ref, k_ref, tw1, tb1, tw2, tb2, aw1, ab1, aw2, ab2,
              bw1, bb1, bw2, bb2, out_ref):
    def mlp(x, w1, b1, w2, b2):
        h = lax.dot_general(x, w1[...], (((1,), (1,)), ((), ())),
                            preferred_element_type=jnp.float32)
        h = jnp.maximum(h + b1[...], 0.0)
        return jnp.sum(h * w2[...], axis=1) + b2[0, 0]

    stat = 8.0 * (jax.nn.sigmoid(mlp(stu_ref[...], tw1, tb1, tw2, tb2)) - 0.5)
    e_diff = jax.nn.sigmoid(mlp(e_ref[...], aw1, ab1, aw2, ab2)) * 2.0
    k_diff = 8.0 * (jax.nn.sigmoid(mlp(k_ref[...], bw1, bb1, bw2, bb2)) - 0.5)
    input_x = jnp.exp(-1.7 * e_diff * (stat - k_diff))
    out_ref[...] = jax.nn.sigmoid(input_x)


def _mlp_fused(stu_rows, e_rows, k_rows,
               a_w1, a_b1, a_w2, a_b2,
               b_w1, b_b1, b_w2, b_b2,
               t_w1, t_b1, t_w2, t_b2):
    B, K = stu_rows.shape
    blk = 2048
    grid = (B // blk,)
    row_spec = pl.BlockSpec((blk, K), lambda i: (i, 0))
    ws = (t_w1, t_b1.reshape(1, -1), t_w2, t_b2.reshape(1, -1),
          a_w1, a_b1.reshape(1, -1), a_w2, a_b2.reshape(1, -1),
          b_w1, b_b1.reshape(1, -1), b_w2, b_b2.reshape(1, -1))
    w_specs = [pl.BlockSpec(memory_space=pltpu.SMEM) if w.size == 1
               else pl.BlockSpec(w.shape, lambda i: (0, 0)) for w in ws]
    return pl.pallas_call(
        _mlp_body,
        grid=grid,
        in_specs=[row_spec, row_spec, row_spec] + w_specs,
        out_specs=pl.BlockSpec((blk,), lambda i: (i,)),
        out_shape=jax.ShapeDtypeStruct((B,), jnp.float32),
    )(stu_rows, e_rows, k_rows, *ws)


def kernel(stu_id, input_exercise, inut_word, inut_format, inut_section,
           inut_wordlen, inut_cefr, input_knowledge_point, student_emb,
           k_difficulty, e_difficulty, k_difficulty_i, e_difficulty_i,
           a_w1, a_b1, a_w2, a_b2, b_w1, b_b1, b_w2, b_b2,
           t_w1, t_b1, t_w2, t_b2):
    stu_rows, k_rows, e_rows = _gather3(
        student_emb, k_difficulty, e_difficulty, stu_id, inut_word)
    return _mlp_fused(stu_rows, e_rows, k_rows,
                      a_w1, a_b1, a_w2, a_b2,
                      b_w1, b_b1, b_w2, b_b2,
                      t_w1, t_b1, t_w2, t_b2)
